# baseline (speedup 1.0000x reference)
import functools

import jax
import jax.numpy as jnp
from jax import lax
from jax.experimental import pallas as pl
from jax.experimental.pallas import tpu as pltpu
from jax.experimental.pallas import tpu_sc as plsc

N = 10000
E = 320000
H = 128
EF = 32
L = 4
G = 64
MAXZ = 100
CUT = 10.0
C = 0.93

NPAD = 10000
NW = 32
_F32 = jnp.float32
_BF16 = jnp.bfloat16

_EW_PERM = [32 * j + (t // 2 if t % 2 == 0 else 16 + t // 2)
            for j in range(4) for t in range(32)]


def _sc_gather_mul_scatter(table, src, dst2d, ew, zeros, chunk):
  B = src.shape[0]
  D = table.shape[1]
  per_w = B // NW
  nch = per_w // chunk
  npairs = nch // 2
  assert per_w % chunk == 0 and chunk % 8 == 0
  mesh = plsc.VectorSubcoreMesh(core_axis_name="c", subcore_axis_name="s",
                                num_cores=2, num_subcores=16)

  @functools.partial(
      pl.kernel, mesh=mesh,
      out_type=jax.ShapeDtypeStruct((2, NPAD, D), _F32),
      compiler_params=pltpu.CompilerParams(needs_layout_passes=False),
      scratch_types=[
          pltpu.VMEM((per_w,), jnp.int32),
          pltpu.VMEM((per_w,), jnp.int32),
          pltpu.VMEM((chunk, D), _F32), pltpu.VMEM((chunk, D), _F32),
          pltpu.VMEM((chunk, D), _BF16), pltpu.VMEM((chunk, D), _BF16),
          pltpu.VMEM_SHARED((NPAD, D), _F32),
          pltpu.SemaphoreType.DMA, pltpu.SemaphoreType.DMA,
          pltpu.SemaphoreType.DMA, pltpu.SemaphoreType.DMA,
      ])
  def k(table_hbm, src_hbm, dst2d_hbm, ew_hbm, zeros_hbm, out_hbm,
        sia, dia, g0, g1, w0, w1, acc_sh, sg0, sg1, sw0, sw1):
    cid = lax.axis_index("c")
    sid = lax.axis_index("s")
    wid = sid * 2 + cid
    r0 = sid * 640

    @pl.when(sid < 15)
    def _():
      pltpu.sync_copy(zeros_hbm.at[pl.ds(r0, 640)], acc_sh.at[pl.ds(r0, 640)])

    @pl.when(sid == 15)
    def _():
      pltpu.sync_copy(zeros_hbm.at[pl.ds(9600, 400)],
                      acc_sh.at[pl.ds(9600, 400)])

    pltpu.sync_copy(src_hbm.at[pl.ds(wid * per_w, per_w)], sia)
    pltpu.sync_copy(dst2d_hbm.at[pl.ds(wid * per_w, per_w)], dia)

    def start(c, g, w, sg, sw):
      pltpu.async_copy(table_hbm.at[sia.at[pl.ds(c * chunk, chunk)]], g, sg)
      pltpu.async_copy(ew_hbm.at[pl.ds(wid * per_w + c * chunk, chunk)], w, sw)

    def finish(c, g, w, sg, sw):
      pltpu.make_async_copy(table_hbm.at[sia.at[pl.ds(0, chunk)]], g, sg).wait()
      pltpu.make_async_copy(ew_hbm.at[pl.ds(0, chunk)], w, sw).wait()

      def row(i, carry2):
        for j in range(4):
          wv = w[i, pl.ds(j * 32, 32)]
          a, b = plsc.unpack(wv, format=plsc.PackFormat.INTERLEAVED)
          sl0 = (i, pl.ds(j * 32, 16))
          sl1 = (i, pl.ds(j * 32 + 16, 16))
          g[sl0] = g[sl0] * a
          g[sl1] = g[sl1] * b
        return carry2

      lax.fori_loop(0, chunk, row, 0)
      pltpu.sync_copy(g, acc_sh.at[dia.at[pl.ds(c * chunk, chunk)]], add=True)

    start(0, g0, w0, sg0, sw0)
    plsc.subcore_barrier()

    def pair(p, carry):
      c0 = p * 2
      start(c0 + 1, g1, w1, sg1, sw1)
      finish(c0, g0, w0, sg0, sw0)

      @pl.when(c0 + 2 < nch)
      def _():
        start(c0 + 2, g0, w0, sg0, sw0)

      finish(c0 + 1, g1, w1, sg1, sw1)
      return carry

    lax.fori_loop(0, npairs, pair, 0)
    if nch % 2 == 1:
      finish(nch - 1, g0, w0, sg0, sw0)
    plsc.subcore_barrier()

    @pl.when(sid < 15)
    def _():
      pltpu.sync_copy(acc_sh.at[pl.ds(r0, 640)],
                      out_hbm.at[cid, pl.ds(r0, 640)])

    @pl.when(sid == 15)
    def _():
      pltpu.sync_copy(acc_sh.at[pl.ds(9600, 400)],
                      out_hbm.at[cid, pl.ds(9600, 400)])

  return k(table, src, dst2d, ew, zeros)


def _sc_edge_dist2(pos4, src, dst, chunk):
  B = src.shape[0]
  per_w = B // NW
  nch = per_w // chunk
  assert per_w % chunk == 0 and chunk % 16 == 0
  mesh = plsc.VectorSubcoreMesh(core_axis_name="c", subcore_axis_name="s",
                                num_cores=2, num_subcores=16)

  @functools.partial(
      pl.kernel, mesh=mesh,
      out_type=jax.ShapeDtypeStruct((B,), _F32),
      compiler_params=pltpu.CompilerParams(needs_layout_passes=False),
      scratch_types=[
          pltpu.VMEM((N * 4,), _F32),
          pltpu.VMEM((chunk,), jnp.int32),
          pltpu.VMEM((chunk,), jnp.int32),
          pltpu.VMEM((chunk,), _F32),
      ])
  def k(pos_hbm, src_hbm, dst_hbm, out_hbm, pos_v, is_v, id_v, o_v):
    wid = lax.axis_index("s") * 2 + lax.axis_index("c")
    pltpu.sync_copy(pos_hbm, pos_v)

    def body(c, carry):
      base = wid * per_w + c * chunk
      pltpu.sync_copy(src_hbm.at[pl.ds(base, chunk)], is_v)
      pltpu.sync_copy(dst_hbm.at[pl.ds(base, chunk)], id_v)

      def inner(j, carry2):
        s_i = is_v[pl.ds(j * 16, 16)] * 4
        d_i = id_v[pl.ds(j * 16, 16)] * 4
        acc = jnp.zeros((16,), _F32)
        for col in range(3):
          xs = plsc.load_gather(pos_v, [s_i + col])
          xd = plsc.load_gather(pos_v, [d_i + col])
          dd = xd - xs
          acc = acc + dd * dd
        o_v[pl.ds(j * 16, 16)] = acc
        return carry2

      lax.fori_loop(0, chunk // 16, inner, 0)
      pltpu.sync_copy(o_v, out_hbm.at[pl.ds(base, chunk)])
      return carry

    lax.fori_loop(0, nch, body, 0)

  return k(pos4, src, dst)


def _silu(x):
  return x * jax.nn.sigmoid(x)


def _tc_edge_feat(d2, efeat, w, b):
  EB = 6400
  grid = E // EB
  gamma = 1.0 / ((CUT / (EF - 1)) ** 2)

  def body(d2_r, ef_r, w_r, b_r, out_r):
    dist = jnp.sqrt(d2_r[...] + 1e-12)
    cen = lax.broadcasted_iota(jnp.int32, (EF, 1), 0).astype(_F32) * (
        CUT / (EF - 1))
    rbf = jnp.exp(-gamma * (dist - cen) ** 2)
    fcut = 0.5 * (jnp.cos(jnp.pi * jnp.clip(dist / CUT, 0.0, 1.0)) + 1.0)
    h = _silu(lax.dot_general(w_r[...], rbf, (((0,), (0,)), ((), ())),
                              preferred_element_type=_F32) + b_r[...])
    out_r[...] = h * fcut + ef_r[...]

  return pl.pallas_call(
      body,
      grid=(grid,),
      in_specs=[
          pl.BlockSpec((1, EB), lambda i: (0, i)),
          pl.BlockSpec((1, EB), lambda i: (0, i)),
          pl.BlockSpec((EF, EF), lambda i: (0, 0)),
          pl.BlockSpec((EF, 1), lambda i: (0, 0)),
      ],
      out_specs=pl.BlockSpec((EF, EB), lambda i: (0, i)),
      out_shape=jax.ShapeDtypeStruct((EF, E), _F32),
  )(d2, efeat, w, b)


def _tc_embed(z3, z_emb, w1, b1, w2, b2):
  NB = 1000
  grid = N // NB

  def body(z_r, emb_r, w1_r, b1_r, w2_r, b2_r, out_r):
    zb = z_r[0, 0, :]
    oh = (zb[:, None] == lax.broadcasted_iota(jnp.int32, (1, MAXZ), 1))
    x = jnp.dot(oh.astype(_F32), emb_r[...], preferred_element_type=_F32)
    x = _silu(jnp.dot(x, w1_r[...], preferred_element_type=_F32) + b1_r[...])
    x = _silu(jnp.dot(x, w2_r[...], preferred_element_type=_F32) + b2_r[...])
    out_r[...] = x

  return pl.pallas_call(
      body,
      grid=(grid,),
      in_specs=[
          pl.BlockSpec((1, 1, NB), lambda i: (i, 0, 0)),
          pl.BlockSpec((MAXZ, H), lambda i: (0, 0)),
          pl.BlockSpec((H, H), lambda i: (0, 0)),
          pl.BlockSpec((1, H), lambda i: (0, 0)),
          pl.BlockSpec((H, H), lambda i: (0, 0)),
          pl.BlockSpec((1, H), lambda i: (0, 0)),
      ],
      out_specs=pl.BlockSpec((NB, H), lambda i: (i, 0)),
      out_shape=jax.ShapeDtypeStruct((N, H), _F32),
  )(z3, z_emb, w1, b1, w2, b2)


def _tc_edgew(efT, w):
  EB = 6400
  grid = E // EB

  def body(ef_r, w_r, out_r):
    out_r[...] = lax.dot_general(ef_r[...], w_r[...], (((0,), (0,)), ((), ())),
                                 preferred_element_type=_F32).astype(_BF16)

  return pl.pallas_call(
      body,
      grid=(grid,),
      in_specs=[
          pl.BlockSpec((EF, EB), lambda i: (0, i)),
          pl.BlockSpec((EF, H), lambda i: (0, 0)),
      ],
      out_specs=pl.BlockSpec((EB, H), lambda i: (i, 0)),
      out_shape=jax.ShapeDtypeStruct((E, H), _BF16),
  )(efT, w)


def _tc_update(p0, p1, scalar, w1, b1, w2, b2):
  NB = 1000
  grid = N // NB

  def body(p0_r, p1_r, s_r, w1_r, b1_r, w2_r, b2_r, out_r):
    agg = p0_r[...] + p1_r[...]
    h = _silu(jnp.dot(agg, w1_r[...], preferred_element_type=_F32) + b1_r[...])
    out_r[...] = s_r[...] + jnp.dot(h, w2_r[...],
                                    preferred_element_type=_F32) + b2_r[...]

  return pl.pallas_call(
      body,
      grid=(grid,),
      in_specs=[
          pl.BlockSpec((NB, H), lambda i: (i, 0)),
          pl.BlockSpec((NB, H), lambda i: (i, 0)),
          pl.BlockSpec((NB, H), lambda i: (i, 0)),
          pl.BlockSpec((H, H), lambda i: (0, 0)),
          pl.BlockSpec((1, H), lambda i: (0, 0)),
          pl.BlockSpec((H, H), lambda i: (0, 0)),
          pl.BlockSpec((1, H), lambda i: (0, 0)),
      ],
      out_specs=pl.BlockSpec((NB, H), lambda i: (i, 0)),
      out_shape=jax.ShapeDtypeStruct((N, H), _F32),
  )(p0, p1, scalar, w1, b1, w2, b2)


def _tc_readout(scalar, batch3, w_ro, b_ro, w_out):
  NB = 1000
  grid = N // NB

  def body(b_r, s_r, wro_r, bro_r, wout_r, out_r, acc):
    i = pl.program_id(0)

    @pl.when(i == 0)
    def _():
      acc[...] = jnp.zeros((G, H), _F32)

    bb = b_r[0, 0, :]
    oh = (bb[:, None] == lax.broadcasted_iota(jnp.int32, (1, G), 1))
    acc[...] += lax.dot_general(oh.astype(_F32), s_r[...],
                                (((0,), (0,)), ((), ())),
                                preferred_element_type=_F32)

    @pl.when(i == grid - 1)
    def _():
      pooled = acc[...] * C
      h = _silu(jnp.dot(pooled, wro_r[...],
                        preferred_element_type=_F32) + bro_r[...])
      out_r[...] = jnp.dot(h, wout_r[...], preferred_element_type=_F32)

  return pl.pallas_call(
      body,
      grid=(grid,),
      in_specs=[
          pl.BlockSpec((1, 1, NB), lambda i: (i, 0, 0)),
          pl.BlockSpec((NB, H), lambda i: (i, 0)),
          pl.BlockSpec((H, H), lambda i: (0, 0)),
          pl.BlockSpec((1, H), lambda i: (0, 0)),
          pl.BlockSpec((H, 1), lambda i: (0, 0)),
      ],
      out_specs=pl.BlockSpec((G, 1), lambda i: (0, 0)),
      out_shape=jax.ShapeDtypeStruct((G, 1), _F32),
      scratch_shapes=[pltpu.VMEM((G, H), _F32)],
  )(batch3, scalar, w_ro, b_ro, w_out)


def kernel(pos, z, edge_index, batch_index, edge_features, subg_node_index,
           subg_node_center_index, subg_edge_index, subg_batch_index,
           subg_edge_features, subg_node_label, z_emb, W_m2g, b_m2g, Wp1, bp1,
           Wp2, bp2, W_ef, W1, b1, W2, b2, W_ro, b_ro, W_out):
  src = edge_index[0]
  dst = edge_index[1]

  pos4 = jnp.pad(pos, ((0, 0), (0, 1))).reshape(N * 4)
  d2 = _sc_edge_dist2(pos4, src, dst, chunk=2000)
  efT = _tc_edge_feat(d2.reshape(1, E), edge_features.reshape(1, E), W_m2g,
                      b_m2g.reshape(EF, 1))

  scalar = _tc_embed(z.reshape(N // 1000, 1, 1000).astype(jnp.int32),
                     z_emb, Wp1, bp1.reshape(1, H), Wp2, bp2.reshape(1, H))

  zeros = jnp.zeros((NPAD, H), _F32)
  perm = jnp.asarray(_EW_PERM, dtype=jnp.int32)
  ews = [_tc_edgew(efT, W_ef[l][:, perm]) for l in range(L)]
  for l in range(L):
    parts = _sc_gather_mul_scatter(scalar, src, dst, ews[l], zeros, chunk=80)
    scalar = _tc_update(parts[0], parts[1], scalar,
                        W1[l], b1[l].reshape(1, H), W2[l], b2[l].reshape(1, H))

  return _tc_readout(scalar, batch_index.reshape(N // 1000, 1, 1000),
                     W_ro, b_ro.reshape(1, H), W_out)


# speedup vs baseline: 1.3121x; 1.3121x over previous
import functools

import jax
import jax.numpy as jnp
from jax import lax
from jax.experimental import pallas as pl
from jax.experimental.pallas import tpu as pltpu
from jax.experimental.pallas import tpu_sc as plsc

N = 10000
E = 320000
H = 128
EF = 32
L = 4
G = 64
MAXZ = 100
CUT = 10.0
C = 0.93

NPAD = 10000
NW = 32
_F32 = jnp.float32


def _sc_gather_mul_scatter(table, src, dst2d, ew, zeros, chunk):
  B = src.shape[0]
  D = table.shape[1]
  per_w = B // NW
  nch = per_w // chunk
  npairs = nch // 2
  assert per_w % chunk == 0 and chunk % 8 == 0
  mesh = plsc.VectorSubcoreMesh(core_axis_name="c", subcore_axis_name="s",
                                num_cores=2, num_subcores=16)

  @functools.partial(
      pl.kernel, mesh=mesh,
      out_type=jax.ShapeDtypeStruct((2, NPAD, D), _F32),
      compiler_params=pltpu.CompilerParams(needs_layout_passes=False),
      scratch_types=[
          pltpu.VMEM((per_w,), jnp.int32),
          pltpu.VMEM((per_w,), jnp.int32),
          pltpu.VMEM((chunk, D), _F32), pltpu.VMEM((chunk, D), _F32),
          pltpu.VMEM((chunk, D), _F32),
          pltpu.VMEM((chunk, D), _F32), pltpu.VMEM((chunk, D), _F32),
          pltpu.VMEM((chunk, D), _F32),
          pltpu.VMEM_SHARED((NPAD, D), _F32),
          pltpu.SemaphoreType.DMA, pltpu.SemaphoreType.DMA,
          pltpu.SemaphoreType.DMA, pltpu.SemaphoreType.DMA,
          pltpu.SemaphoreType.DMA, pltpu.SemaphoreType.DMA,
      ])
  def k(table_hbm, src_hbm, dst2d_hbm, ew_hbm, zeros_hbm, out_hbm,
        sia, dia, g0, g1, g2, w0, w1, w2, acc_sh,
        sg0, sg1, sg2, sw0, sw1, sw2):
    cid = lax.axis_index("c")
    sid = lax.axis_index("s")
    wid = sid * 2 + cid
    r0 = sid * 640

    @pl.when(sid < 15)
    def _():
      pltpu.sync_copy(zeros_hbm.at[pl.ds(r0, 640)], acc_sh.at[pl.ds(r0, 640)])

    @pl.when(sid == 15)
    def _():
      pltpu.sync_copy(zeros_hbm.at[pl.ds(9600, 400)],
                      acc_sh.at[pl.ds(9600, 400)])

    pltpu.sync_copy(src_hbm.at[pl.ds(wid * per_w, per_w)], sia)
    pltpu.sync_copy(dst2d_hbm.at[pl.ds(wid * per_w, per_w)], dia)

    def start(c, g, w, sg, sw):
      pltpu.async_copy(table_hbm.at[sia.at[pl.ds(c * chunk, chunk)]], g, sg)
      pltpu.async_copy(ew_hbm.at[pl.ds(wid * per_w + c * chunk, chunk)], w, sw)

    def finish(c, g, w, sg, sw):
      pltpu.make_async_copy(table_hbm.at[sia.at[pl.ds(0, chunk)]], g, sg).wait()
      pltpu.make_async_copy(ew_hbm.at[pl.ds(0, chunk)], w, sw).wait()

      def row(i, carry2):
        for j in range(8):
          sl = (i, pl.ds(j * 16, 16))
          g[sl] = g[sl] * w[sl]
        return carry2

      lax.fori_loop(0, chunk, row, 0)
      pltpu.sync_copy(g, acc_sh.at[dia.at[pl.ds(c * chunk, chunk)]], add=True)

    bufs = ((g0, w0, sg0, sw0), (g1, w1, sg1, sw1), (g2, w2, sg2, sw2))
    start(0, *bufs[0])
    start(1, *bufs[1])
    start(2, *bufs[2])
    plsc.subcore_barrier()

    ntrip = (nch - 1) // 3

    def trip(p, carry):
      c0 = p * 3
      for t in range(3):
        finish(c0 + t, *bufs[t])

        @pl.when(c0 + t + 3 < nch)
        def _():
          start(c0 + t + 3, *bufs[t])
      return carry

    lax.fori_loop(0, ntrip, trip, 0)
    for c in range(3 * ntrip, nch):
      finish(c, *bufs[c % 3])
    plsc.subcore_barrier()

    @pl.when(sid < 15)
    def _():
      pltpu.sync_copy(acc_sh.at[pl.ds(r0, 640)],
                      out_hbm.at[cid, pl.ds(r0, 640)])

    @pl.when(sid == 15)
    def _():
      pltpu.sync_copy(acc_sh.at[pl.ds(9600, 400)],
                      out_hbm.at[cid, pl.ds(9600, 400)])

  return k(table, src, dst2d, ew, zeros)


def _sc_edge_dist2(pos4, src, dst, chunk):
  B = src.shape[0]
  per_w = B // NW
  nch = per_w // chunk
  assert per_w % chunk == 0 and chunk % 16 == 0
  mesh = plsc.VectorSubcoreMesh(core_axis_name="c", subcore_axis_name="s",
                                num_cores=2, num_subcores=16)

  @functools.partial(
      pl.kernel, mesh=mesh,
      out_type=jax.ShapeDtypeStruct((B,), _F32),
      compiler_params=pltpu.CompilerParams(needs_layout_passes=False),
      scratch_types=[
          pltpu.VMEM((N * 4,), _F32),
          pltpu.VMEM((chunk,), jnp.int32),
          pltpu.VMEM((chunk,), jnp.int32),
          pltpu.VMEM((chunk,), _F32),
      ])
  def k(pos_hbm, src_hbm, dst_hbm, out_hbm, pos_v, is_v, id_v, o_v):
    wid = lax.axis_index("s") * 2 + lax.axis_index("c")
    pltpu.sync_copy(pos_hbm, pos_v)

    def body(c, carry):
      base = wid * per_w + c * chunk
      pltpu.sync_copy(src_hbm.at[pl.ds(base, chunk)], is_v)
      pltpu.sync_copy(dst_hbm.at[pl.ds(base, chunk)], id_v)

      def inner(j, carry2):
        s_i = is_v[pl.ds(j * 16, 16)] * 4
        d_i = id_v[pl.ds(j * 16, 16)] * 4
        acc = jnp.zeros((16,), _F32)
        for col in range(3):
          xs = plsc.load_gather(pos_v, [s_i + col])
          xd = plsc.load_gather(pos_v, [d_i + col])
          dd = xd - xs
          acc = acc + dd * dd
        o_v[pl.ds(j * 16, 16)] = acc
        return carry2

      lax.fori_loop(0, chunk // 16, inner, 0)
      pltpu.sync_copy(o_v, out_hbm.at[pl.ds(base, chunk)])
      return carry

    lax.fori_loop(0, nch, body, 0)

  return k(pos4, src, dst)


def _silu(x):
  return x * jax.nn.sigmoid(x)


def _tc_edge_feat(d2, efeat, w, b):
  EB = 6400
  grid = E // EB
  gamma = 1.0 / ((CUT / (EF - 1)) ** 2)

  def body(d2_r, ef_r, w_r, b_r, out_r):
    dist = jnp.sqrt(d2_r[...] + 1e-12)
    cen = lax.broadcasted_iota(jnp.int32, (EF, 1), 0).astype(_F32) * (
        CUT / (EF - 1))
    rbf = jnp.exp(-gamma * (dist - cen) ** 2)
    fcut = 0.5 * (jnp.cos(jnp.pi * jnp.clip(dist / CUT, 0.0, 1.0)) + 1.0)
    h = _silu(lax.dot_general(w_r[...], rbf, (((0,), (0,)), ((), ())),
                              preferred_element_type=_F32) + b_r[...])
    out_r[...] = h * fcut + ef_r[...]

  return pl.pallas_call(
      body,
      grid=(grid,),
      in_specs=[
          pl.BlockSpec((1, EB), lambda i: (0, i)),
          pl.BlockSpec((1, EB), lambda i: (0, i)),
          pl.BlockSpec((EF, EF), lambda i: (0, 0)),
          pl.BlockSpec((EF, 1), lambda i: (0, 0)),
      ],
      out_specs=pl.BlockSpec((EF, EB), lambda i: (0, i)),
      out_shape=jax.ShapeDtypeStruct((EF, E), _F32),
  )(d2, efeat, w, b)


def _tc_embed(z3, z_emb, w1, b1, w2, b2):
  NB = 1000
  grid = N // NB

  def body(z_r, emb_r, w1_r, b1_r, w2_r, b2_r, out_r):
    zb = z_r[0, 0, :]
    oh = (zb[:, None] == lax.broadcasted_iota(jnp.int32, (1, MAXZ), 1))
    x = jnp.dot(oh.astype(_F32), emb_r[...], preferred_element_type=_F32)
    x = _silu(jnp.dot(x, w1_r[...], preferred_element_type=_F32) + b1_r[...])
    x = _silu(jnp.dot(x, w2_r[...], preferred_element_type=_F32) + b2_r[...])
    out_r[...] = x

  return pl.pallas_call(
      body,
      grid=(grid,),
      in_specs=[
          pl.BlockSpec((1, 1, NB), lambda i: (i, 0, 0)),
          pl.BlockSpec((MAXZ, H), lambda i: (0, 0)),
          pl.BlockSpec((H, H), lambda i: (0, 0)),
          pl.BlockSpec((1, H), lambda i: (0, 0)),
          pl.BlockSpec((H, H), lambda i: (0, 0)),
          pl.BlockSpec((1, H), lambda i: (0, 0)),
      ],
      out_specs=pl.BlockSpec((NB, H), lambda i: (i, 0)),
      out_shape=jax.ShapeDtypeStruct((N, H), _F32),
  )(z3, z_emb, w1, b1, w2, b2)


def _tc_edgew(efT, w):
  EB = 6400
  grid = E // EB

  def body(ef_r, w_r, out_r):
    out_r[...] = lax.dot_general(ef_r[...], w_r[...], (((0,), (0,)), ((), ())),
                                 preferred_element_type=_F32)

  return pl.pallas_call(
      body,
      grid=(grid,),
      in_specs=[
          pl.BlockSpec((EF, EB), lambda i: (0, i)),
          pl.BlockSpec((EF, H), lambda i: (0, 0)),
      ],
      out_specs=pl.BlockSpec((EB, H), lambda i: (i, 0)),
      out_shape=jax.ShapeDtypeStruct((E, H), _F32),
  )(efT, w)


def _tc_update(p0, p1, scalar, w1, b1, w2, b2):
  NB = 1000
  grid = N // NB

  def body(p0_r, p1_r, s_r, w1_r, b1_r, w2_r, b2_r, out_r):
    agg = p0_r[...] + p1_r[...]
    h = _silu(jnp.dot(agg, w1_r[...], preferred_element_type=_F32) + b1_r[...])
    out_r[...] = s_r[...] + jnp.dot(h, w2_r[...],
                                    preferred_element_type=_F32) + b2_r[...]

  return pl.pallas_call(
      body,
      grid=(grid,),
      in_specs=[
          pl.BlockSpec((NB, H), lambda i: (i, 0)),
          pl.BlockSpec((NB, H), lambda i: (i, 0)),
          pl.BlockSpec((NB, H), lambda i: (i, 0)),
          pl.BlockSpec((H, H), lambda i: (0, 0)),
          pl.BlockSpec((1, H), lambda i: (0, 0)),
          pl.BlockSpec((H, H), lambda i: (0, 0)),
          pl.BlockSpec((1, H), lambda i: (0, 0)),
      ],
      out_specs=pl.BlockSpec((NB, H), lambda i: (i, 0)),
      out_shape=jax.ShapeDtypeStruct((N, H), _F32),
  )(p0, p1, scalar, w1, b1, w2, b2)


def _tc_readout(scalar, batch3, w_ro, b_ro, w_out):
  NB = 1000
  grid = N // NB

  def body(b_r, s_r, wro_r, bro_r, wout_r, out_r, acc):
    i = pl.program_id(0)

    @pl.when(i == 0)
    def _():
      acc[...] = jnp.zeros((G, H), _F32)

    bb = b_r[0, 0, :]
    oh = (bb[:, None] == lax.broadcasted_iota(jnp.int32, (1, G), 1))
    acc[...] += lax.dot_general(oh.astype(_F32), s_r[...],
                                (((0,), (0,)), ((), ())),
                                preferred_element_type=_F32)

    @pl.when(i == grid - 1)
    def _():
      pooled = acc[...] * C
      h = _silu(jnp.dot(pooled, wro_r[...],
                        preferred_element_type=_F32) + bro_r[...])
      out_r[...] = jnp.dot(h, wout_r[...], preferred_element_type=_F32)

  return pl.pallas_call(
      body,
      grid=(grid,),
      in_specs=[
          pl.BlockSpec((1, 1, NB), lambda i: (i, 0, 0)),
          pl.BlockSpec((NB, H), lambda i: (i, 0)),
          pl.BlockSpec((H, H), lambda i: (0, 0)),
          pl.BlockSpec((1, H), lambda i: (0, 0)),
          pl.BlockSpec((H, 1), lambda i: (0, 0)),
      ],
      out_specs=pl.BlockSpec((G, 1), lambda i: (0, 0)),
      out_shape=jax.ShapeDtypeStruct((G, 1), _F32),
      scratch_shapes=[pltpu.VMEM((G, H), _F32)],
  )(batch3, scalar, w_ro, b_ro, w_out)


def kernel(pos, z, edge_index, batch_index, edge_features, subg_node_index,
           subg_node_center_index, subg_edge_index, subg_batch_index,
           subg_edge_features, subg_node_label, z_emb, W_m2g, b_m2g, Wp1, bp1,
           Wp2, bp2, W_ef, W1, b1, W2, b2, W_ro, b_ro, W_out):
  src = edge_index[0]
  dst = edge_index[1]

  pos4 = jnp.pad(pos, ((0, 0), (0, 1))).reshape(N * 4)
  d2 = _sc_edge_dist2(pos4, src, dst, chunk=2000)
  efT = _tc_edge_feat(d2.reshape(1, E), edge_features.reshape(1, E), W_m2g,
                      b_m2g.reshape(EF, 1))

  scalar = _tc_embed(z.reshape(N // 1000, 1, 1000).astype(jnp.int32),
                     z_emb, Wp1, bp1.reshape(1, H), Wp2, bp2.reshape(1, H))

  zeros = jnp.zeros((NPAD, H), _F32)
  ews = [_tc_edgew(efT, W_ef[l]) for l in range(L)]
  for l in range(L):
    parts = _sc_gather_mul_scatter(scalar, src, dst, ews[l], zeros, chunk=40)
    scalar = _tc_update(parts[0], parts[1], scalar,
                        W1[l], b1[l].reshape(1, H), W2[l], b2[l].reshape(1, H))

  return _tc_readout(scalar, batch_index.reshape(N // 1000, 1, 1000),
                     W_ro, b_ro.reshape(1, H), W_out)
